# Initial kernel scaffold; baseline (speedup 1.0000x reference)
import functools

import jax
import jax.numpy as jnp
from jax import lax
from jax.experimental import pallas as pl
from jax.experimental.pallas import tpu as pltpu
from jax.experimental.pallas import tpu_sc as plsc

NUM_FIELDS = 26
VOCAB = 100000
DIM = 16
BATCH = 16384

NC, NS, L = 2, 16, 16
NW = NC * NS
TOTAL = BATCH * NUM_FIELDS
PER_W = TOTAL // NW
CHUNK = 128
N_CHUNKS = PER_W // CHUNK
VECS = PER_W // L

_mesh = plsc.VectorSubcoreMesh(core_axis_name="c", subcore_axis_name="s")


@functools.partial(
    pl.kernel,
    mesh=_mesh,
    out_type=jax.ShapeDtypeStruct((TOTAL, DIM), jnp.float32),
    scratch_types=[
        pltpu.VMEM((PER_W,), jnp.int32),
        pltpu.VMEM((PER_W,), jnp.int32),
        pltpu.VMEM((2, CHUNK, DIM), jnp.float32),
        pltpu.SemaphoreType.DMA,
        pltpu.SemaphoreType.DMA,
    ],
)
def _embed(feat_hbm, table_hbm, out_hbm, feat_v, idx_v, rows_v, gsem, ssem):
    wid = lax.axis_index("s") * NC + lax.axis_index("c")
    base = wid * PER_W
    pltpu.sync_copy(feat_hbm.at[pl.ds(base, PER_W)], feat_v)

    lanes = lax.iota(jnp.int32, L)

    def compute_idx(c, _):
        pos = c * L + lanes
        col = lax.rem(pos, NUM_FIELDS)
        idx_v[pl.ds(c * L, L)] = feat_v[pl.ds(c * L, L)] + col * VOCAB
        return 0

    lax.fori_loop(0, VECS, compute_idx, 0)

    def do_chunk(j, _):
        buf = rows_v.at[0]
        pltpu.async_copy(
            table_hbm.at[idx_v.at[pl.ds(j * CHUNK, CHUNK)]], buf, gsem
        ).wait()
        pltpu.sync_copy(buf, out_hbm.at[pl.ds(base + j * CHUNK, CHUNK)])
        return 0

    lax.fori_loop(0, N_CHUNKS, do_chunk, 0)


def kernel(features, tables):
    feat = features.reshape(-1).astype(jnp.int32)
    table = tables.reshape(NUM_FIELDS * VOCAB, DIM)
    out = _embed(feat, table)
    return out.reshape(BATCH, NUM_FIELDS * DIM)


# speedup vs baseline: 1.0842x; 1.0842x over previous
import functools

import jax
import jax.numpy as jnp
from jax import lax
from jax.experimental import pallas as pl
from jax.experimental.pallas import tpu as pltpu
from jax.experimental.pallas import tpu_sc as plsc

NUM_FIELDS = 26
VOCAB = 100000
DIM = 16
BATCH = 16384

NC, NS, L = 2, 16, 16
NW = NC * NS
TOTAL = BATCH * NUM_FIELDS
PER_W = TOTAL // NW
CHUNK = 128
N_CHUNKS = PER_W // CHUNK
VECS = PER_W // L

_mesh = plsc.VectorSubcoreMesh(core_axis_name="c", subcore_axis_name="s")


@functools.partial(
    pl.kernel,
    mesh=_mesh,
    compiler_params=pltpu.CompilerParams(use_tc_tiling_on_sc=False),
    out_type=jax.ShapeDtypeStruct((TOTAL, DIM), jnp.float32),
    scratch_types=[
        pltpu.VMEM((PER_W,), jnp.int32),
        pltpu.VMEM((PER_W,), jnp.int32),
        pltpu.VMEM((2, CHUNK, DIM), jnp.float32),
        pltpu.SemaphoreType.DMA,
        pltpu.SemaphoreType.DMA,
    ],
)
def _embed(feat_hbm, table_hbm, out_hbm, feat_v, idx_v, rows_v, gsem, ssem):
    wid = lax.axis_index("s") * NC + lax.axis_index("c")
    base = wid * PER_W
    pltpu.sync_copy(feat_hbm.at[pl.ds(base, PER_W)], feat_v)

    lanes = lax.iota(jnp.int32, L)

    def compute_idx(c, _):
        pos = c * L + lanes
        col = lax.rem(pos, NUM_FIELDS)
        idx_v[pl.ds(c * L, L)] = feat_v[pl.ds(c * L, L)] + col * VOCAB
        return 0

    lax.fori_loop(0, VECS, compute_idx, 0)

    def do_chunk(j, _):
        buf = rows_v.at[0]
        pltpu.async_copy(
            table_hbm.at[idx_v.at[pl.ds(j * CHUNK, CHUNK)]], buf, gsem
        ).wait()
        pltpu.sync_copy(buf, out_hbm.at[pl.ds(base + j * CHUNK, CHUNK)])
        return 0

    lax.fori_loop(0, N_CHUNKS, do_chunk, 0)


def kernel(features, tables):
    feat = features.reshape(-1).astype(jnp.int32)
    table = tables.reshape(NUM_FIELDS * VOCAB, DIM)
    out = _embed(feat, table)
    return out.reshape(BATCH, NUM_FIELDS * DIM)


# speedup vs baseline: 1.1516x; 1.0622x over previous
import functools

import jax
import jax.numpy as jnp
from jax import lax
from jax.experimental import pallas as pl
from jax.experimental.pallas import tpu as pltpu
from jax.experimental.pallas import tpu_sc as plsc

NUM_FIELDS = 26
VOCAB = 100000
DIM = 16
BATCH = 16384

NC, NS, L = 2, 16, 16
NW = NC * NS
TOTAL = BATCH * NUM_FIELDS
PER_W = TOTAL // NW
CHUNK = 128
K = 13
STAGES = PER_W // (CHUNK * K)
VECS = PER_W // L

_mesh = plsc.VectorSubcoreMesh(core_axis_name="c", subcore_axis_name="s")


@functools.partial(
    pl.kernel,
    mesh=_mesh,
    compiler_params=pltpu.CompilerParams(use_tc_tiling_on_sc=False),
    out_type=jax.ShapeDtypeStruct((TOTAL, DIM), jnp.float32),
    scratch_types=[
        pltpu.VMEM((PER_W,), jnp.int32),
        pltpu.VMEM((PER_W,), jnp.int32),
        pltpu.VMEM((K, CHUNK, DIM), jnp.float32),
        pltpu.VMEM((K, CHUNK, DIM), jnp.float32),
        pltpu.SemaphoreType.DMA,
        pltpu.SemaphoreType.DMA,
        pltpu.SemaphoreType.DMA,
        pltpu.SemaphoreType.DMA,
    ],
)
def _embed(feat_hbm, table_hbm, out_hbm, feat_v, idx_v, rows_a, rows_b,
           gsem_a, gsem_b, ssem_a, ssem_b):
    wid = lax.axis_index("s") * NC + lax.axis_index("c")
    base = wid * PER_W
    pltpu.sync_copy(feat_hbm.at[pl.ds(base, PER_W)], feat_v)

    lanes = lax.iota(jnp.int32, L)

    def compute_idx(c, _):
        pos = c * L + lanes
        col = lax.rem(pos, NUM_FIELDS)
        idx_v[pl.ds(c * L, L)] = feat_v[pl.ds(c * L, L)] + col * VOCAB
        return 0

    lax.fori_loop(0, VECS, compute_idx, 0)

    def fire_gathers(stage, rows, sem):
        for b in range(K):
            j = stage * K + b
            pltpu.async_copy(
                table_hbm.at[idx_v.at[pl.ds(j * CHUNK, CHUNK)]],
                rows.at[b], sem)

    def wait_gathers(rows, sem):
        for b in range(K):
            pltpu.make_async_copy(
                table_hbm.at[idx_v.at[pl.ds(0, CHUNK)]],
                rows.at[b], sem).wait()

    def fire_stores(stage, rows, sem):
        for b in range(K):
            j = stage * K + b
            pltpu.async_copy(
                rows.at[b], out_hbm.at[pl.ds(base + j * CHUNK, CHUNK)], sem)

    def wait_stores(rows, sem):
        for b in range(K):
            pltpu.make_async_copy(
                rows.at[b], out_hbm.at[pl.ds(base, CHUNK)], sem).wait()

    fire_gathers(0, rows_a, gsem_a)

    def body(i, _):
        sa = 2 * i
        sb = 2 * i + 1
        pl.when(i > 0)(lambda: wait_stores(rows_b, ssem_b))
        fire_gathers(sb, rows_b, gsem_b)
        wait_gathers(rows_a, gsem_a)
        fire_stores(sa, rows_a, ssem_a)
        wait_gathers(rows_b, gsem_b)
        fire_stores(sb, rows_b, ssem_b)
        wait_stores(rows_a, ssem_a)
        pl.when(i < STAGES // 2 - 1)(
            lambda: fire_gathers(sa + 2, rows_a, gsem_a))
        return 0

    lax.fori_loop(0, STAGES // 2, body, 0)
    wait_stores(rows_b, ssem_b)


def kernel(features, tables):
    feat = features.reshape(-1).astype(jnp.int32)
    table = tables.reshape(NUM_FIELDS * VOCAB, DIM)
    out = _embed(feat, table)
    return out.reshape(BATCH, NUM_FIELDS * DIM)


# speedup vs baseline: 1.1523x; 1.0006x over previous
import functools

import jax
import jax.numpy as jnp
from jax import lax
from jax.experimental import pallas as pl
from jax.experimental.pallas import tpu as pltpu
from jax.experimental.pallas import tpu_sc as plsc

NUM_FIELDS = 26
VOCAB = 100000
DIM = 16
BATCH = 16384

NC, NS, L = 2, 16, 16
NW = NC * NS
ROWS_W = BATCH // NW
CHUNK = 128
K = ROWS_W // CHUNK

_mesh = plsc.VectorSubcoreMesh(core_axis_name="c", subcore_axis_name="s")


@functools.partial(
    pl.kernel,
    mesh=_mesh,
    compiler_params=pltpu.CompilerParams(use_tc_tiling_on_sc=False),
    out_type=jax.ShapeDtypeStruct((BATCH, NUM_FIELDS * DIM), jnp.float32),
    scratch_types=[
        pltpu.VMEM((NUM_FIELDS, ROWS_W), jnp.int32),
        pltpu.VMEM((K, CHUNK, DIM), jnp.float32),
        pltpu.VMEM((K, CHUNK, DIM), jnp.float32),
        pltpu.SemaphoreType.DMA,
        pltpu.SemaphoreType.DMA,
        pltpu.SemaphoreType.DMA,
        pltpu.SemaphoreType.DMA,
    ],
)
def _embed(feat_hbm, table_hbm, out_hbm, idx_v, rows_a, rows_b,
           gsem_a, gsem_b, ssem_a, ssem_b):
    wid = lax.axis_index("s") * NC + lax.axis_index("c")
    bbase = wid * ROWS_W
    pltpu.sync_copy(feat_hbm.at[:, pl.ds(bbase, ROWS_W)], idx_v)

    def fire_gathers(f, rows, sem):
        for c in range(K):
            pltpu.async_copy(
                table_hbm.at[f].at[idx_v.at[f, pl.ds(c * CHUNK, CHUNK)]],
                rows.at[c], sem)

    def wait_gathers(rows, sem):
        for c in range(K):
            pltpu.make_async_copy(
                table_hbm.at[0].at[idx_v.at[0, pl.ds(0, CHUNK)]],
                rows.at[c], sem).wait()

    def fire_stores(f, rows, sem):
        for c in range(K):
            pltpu.async_copy(
                rows.at[c],
                out_hbm.at[pl.ds(bbase + c * CHUNK, CHUNK),
                           pl.ds(f * DIM, DIM)], sem)

    def wait_stores(rows, sem):
        for c in range(K):
            pltpu.make_async_copy(
                rows.at[c],
                out_hbm.at[pl.ds(bbase, CHUNK), pl.ds(0, DIM)], sem).wait()

    fire_gathers(0, rows_a, gsem_a)

    def body(i, _):
        fa = 2 * i
        fb = 2 * i + 1
        pl.when(i > 0)(lambda: wait_stores(rows_b, ssem_b))
        fire_gathers(fb, rows_b, gsem_b)
        wait_gathers(rows_a, gsem_a)
        fire_stores(fa, rows_a, ssem_a)
        wait_gathers(rows_b, gsem_b)
        fire_stores(fb, rows_b, ssem_b)
        wait_stores(rows_a, ssem_a)
        pl.when(i < NUM_FIELDS // 2 - 1)(
            lambda: fire_gathers(fa + 2, rows_a, gsem_a))
        return 0

    lax.fori_loop(0, NUM_FIELDS // 2, body, 0)
    wait_stores(rows_b, ssem_b)


def kernel(features, tables):
    feat_t = features.T.astype(jnp.int32)
    return _embed(feat_t, tables)


# speedup vs baseline: 2.2462x; 1.9493x over previous
import functools

import jax
import jax.numpy as jnp
from jax import lax
from jax.experimental import pallas as pl
from jax.experimental.pallas import tpu as pltpu
from jax.experimental.pallas import tpu_sc as plsc

NUM_FIELDS = 26
VOCAB = 100000
DIM = 16
BATCH = 16384

NC, NS, L = 2, 16, 16
NW = NC * NS
NROWS = NUM_FIELDS * DIM
SPLITS = 13
ROWS_S = NROWS // SPLITS

_mesh = plsc.VectorSubcoreMesh(core_axis_name="c", subcore_axis_name="s")


def _make_embed(s):
    @functools.partial(
        pl.kernel,
        mesh=_mesh,
        compiler_params=pltpu.CompilerParams(use_tc_tiling_on_sc=False),
        out_type=jax.ShapeDtypeStruct((ROWS_S, BATCH), jnp.float32),
        scratch_types=[
            pltpu.VMEM((BATCH,), jnp.int32),
            pltpu.VMEM((BATCH,), jnp.float32),
            pltpu.SemaphoreType.DMA,
            pltpu.SemaphoreType.DMA,
            pltpu.SemaphoreType.DMA,
        ],
    )
    def _embed(feat_hbm, tbl_hbm, out_hbm, idx_v, og, isem, gsem, ssem):
        wid = lax.axis_index("s") * NC + lax.axis_index("c")
        f = lax.shift_right_logical(s * ROWS_S + wid, 4)
        pltpu.async_copy(feat_hbm.at[f], idx_v, isem).wait()
        pltpu.async_copy(tbl_hbm.at[wid].at[idx_v], og, gsem).wait()
        pltpu.async_copy(og, out_hbm.at[wid], ssem).wait()

    return _embed


def kernel(features, tables):
    feat_t = features.T.astype(jnp.int32)
    tbl = tables.transpose(0, 2, 1).reshape(NROWS, VOCAB)
    outs = [
        _make_embed(s)(feat_t, tbl[s * ROWS_S:(s + 1) * ROWS_S])
        for s in range(SPLITS)
    ]
    out_t = jnp.concatenate(outs, axis=0)
    return out_t.T


# speedup vs baseline: 2.2531x; 1.0031x over previous
import functools

import jax
import jax.numpy as jnp
from jax import lax
from jax.experimental import pallas as pl
from jax.experimental.pallas import tpu as pltpu
from jax.experimental.pallas import tpu_sc as plsc

NUM_FIELDS = 26
VOCAB = 100000
DIM = 16
BATCH = 16384

NC, NS, L = 2, 16, 16
NW = NC * NS
NROWS = NUM_FIELDS * DIM
SPLITS = 13
ROWS_S = NROWS // SPLITS

_mesh = plsc.VectorSubcoreMesh(core_axis_name="c", subcore_axis_name="s")


def _make_embed(s):
    @functools.partial(
        pl.kernel,
        mesh=_mesh,
        compiler_params=pltpu.CompilerParams(use_tc_tiling_on_sc=False),
        out_type=jax.ShapeDtypeStruct((ROWS_S, BATCH), jnp.float32),
        scratch_types=[
            pltpu.VMEM((BATCH,), jnp.int32),
            pltpu.VMEM((2, BATCH // 2), jnp.float32),
            pltpu.SemaphoreType.DMA,
            pltpu.SemaphoreType.DMA,
            pltpu.SemaphoreType.DMA,
        ],
    )
    def _embed(feat_hbm, tbl_hbm, out_hbm, idx_v, og, isem, gsem, ssem):
        wid = lax.axis_index("s") * NC + lax.axis_index("c")
        f = lax.shift_right_logical(s * ROWS_S + wid, 4)
        half = BATCH // 2
        pltpu.async_copy(feat_hbm.at[f], idx_v, isem).wait()
        row = tbl_hbm.at[wid]
        g0 = pltpu.async_copy(row.at[idx_v.at[pl.ds(0, half)]], og.at[0], gsem)
        g1 = pltpu.async_copy(row.at[idx_v.at[pl.ds(half, half)]], og.at[1],
                              gsem)
        g0.wait()
        s0 = pltpu.async_copy(og.at[0], out_hbm.at[wid, pl.ds(0, half)], ssem)
        g1.wait()
        s1 = pltpu.async_copy(og.at[1], out_hbm.at[wid, pl.ds(half, half)],
                              ssem)
        s0.wait()
        s1.wait()

    return _embed


def kernel(features, tables):
    feat_t = features.T.astype(jnp.int32)
    tbl = tables.transpose(0, 2, 1).reshape(NROWS, VOCAB)
    outs = [
        _make_embed(s)(feat_t, tbl[s * ROWS_S:(s + 1) * ROWS_S])
        for s in range(SPLITS)
    ]
    out_t = jnp.concatenate(outs, axis=0)
    return out_t.T


# speedup vs baseline: 2.4204x; 1.0742x over previous
import functools

import jax
import jax.numpy as jnp
from jax import lax
from jax.experimental import pallas as pl
from jax.experimental.pallas import tpu as pltpu
from jax.experimental.pallas import tpu_sc as plsc

NUM_FIELDS = 26
VOCAB = 100000
DIM = 16
BATCH = 16384

NC, NS, L = 2, 16, 16
NW = NC * NS
NROWS = NUM_FIELDS * DIM

SPLITS = [(0, 128, 4), (128, 96, 3), (224, 96, 3), (320, 96, 3)]

_mesh = plsc.VectorSubcoreMesh(core_axis_name="c", subcore_axis_name="s")


def _make_embed(r0, rs, pw):
    @functools.partial(
        pl.kernel,
        mesh=_mesh,
        compiler_params=pltpu.CompilerParams(use_tc_tiling_on_sc=False),
        out_type=jax.ShapeDtypeStruct((rs, BATCH), jnp.float32),
        scratch_types=[
            pltpu.VMEM((2, BATCH), jnp.int32),
            pltpu.VMEM((BATCH,), jnp.float32),
            pltpu.VMEM((BATCH,), jnp.float32),
            pltpu.SemaphoreType.DMA,
            pltpu.SemaphoreType.DMA,
            pltpu.SemaphoreType.DMA,
            pltpu.SemaphoreType.DMA,
            pltpu.SemaphoreType.DMA,
        ],
    )
    def _embed(feat_hbm, tbl_hbm, out_hbm, idx_v, ga, gb,
               isem, gsem_a, gsem_b, ssem_a, ssem_b):
        wid = lax.axis_index("s") * NC + lax.axis_index("c")
        p0 = wid * pw
        f0 = lax.shift_right_logical(r0 + p0, 4)
        f1 = lax.min(f0 + 1, NUM_FIELDS - 1)

        pltpu.async_copy(feat_hbm.at[f0], idx_v.at[0], isem)
        pltpu.async_copy(feat_hbm.at[f1], idx_v.at[1], isem)

        def idx_of(p):
            return idx_v.at[lax.shift_right_logical(r0 + p, 4) - f0]

        def fire_gather(p, buf, sem):
            pltpu.async_copy(tbl_hbm.at[p].at[idx_of(p)], buf, sem)

        def wait_gather(buf, sem):
            pltpu.make_async_copy(
                tbl_hbm.at[0].at[idx_v.at[0]], buf, sem).wait()

        def fire_store(p, buf, sem):
            pltpu.async_copy(buf, out_hbm.at[p], sem)

        def wait_store(buf, sem):
            pltpu.make_async_copy(buf, out_hbm.at[0], sem).wait()

        pltpu.make_async_copy(feat_hbm.at[0], idx_v.at[0], isem).wait()
        pltpu.make_async_copy(feat_hbm.at[0], idx_v.at[1], isem).wait()
        fire_gather(p0, ga, gsem_a)

        def body(i, _):
            pa = p0 + 2 * i
            pb = pa + 1
            pl.when(i > 0)(lambda: wait_store(gb, ssem_b))
            pl.when(2 * i + 1 < pw)(lambda: fire_gather(pb, gb, gsem_b))
            wait_gather(ga, gsem_a)
            fire_store(pa, ga, ssem_a)

            @pl.when(2 * i + 1 < pw)
            def _():
                wait_gather(gb, gsem_b)
                fire_store(pb, gb, ssem_b)

            wait_store(ga, ssem_a)

            @pl.when(2 * i + 2 < pw)
            def _():
                fire_gather(pa + 2, ga, gsem_a)

            return 0

        lax.fori_loop(0, (pw + 1) // 2, body, 0)
        if pw % 2 == 0:
            wait_store(gb, ssem_b)

    return _embed


def kernel(features, tables):
    feat_t = features.T.astype(jnp.int32)
    tbl = tables.transpose(0, 2, 1).reshape(NROWS, VOCAB)
    outs = [
        _make_embed(r0, rs, pw)(feat_t, tbl[r0:r0 + rs])
        for (r0, rs, pw) in SPLITS
    ]
    out_t = jnp.concatenate(outs, axis=0)
    return out_t.T
